# baseline (speedup 1.0000x reference)
import jax
import jax.numpy as jnp
from jax.experimental import pallas as pl

HEADS = 32
BUCKET_SIZE = 64
MAX_BUCKETS = 64
DIM = 256
TEMPERATURE = 0.7


T_SPLIT = 2
ROWS = MAX_BUCKETS // T_SPLIT


def _body(q_ref, k_ref, w_ref, o_ref):
    qs = jnp.sum(q_ref[0].reshape(ROWS, BUCKET_SIZE, 128), axis=1)
    ks = jnp.sum(k_ref[0].reshape(ROWS, BUCKET_SIZE, 128), axis=1)
    w = w_ref[0, 0]
    r = jnp.dot(qs, w[:128, :], preferred_element_type=jnp.float32)
    r = r + jnp.dot(ks, w[128:, :], preferred_element_type=jnp.float32)
    r = jnp.maximum(r, 0.0)

    m = jnp.max(r, axis=-1, keepdims=True)
    iota = jax.lax.broadcasted_iota(jnp.int32, r.shape, 1)
    idx = jnp.min(jnp.where(r == m, iota, MAX_BUCKETS), axis=-1, keepdims=True)
    denom = jnp.sum(jnp.exp((r - m) / TEMPERATURE), axis=-1, keepdims=True)
    val = 1.0 / denom
    o_ref[0] = jnp.where(iota == idx, val, 0.0)


def kernel(q, k, linear, topk):
    bh = q.shape[0]
    tchunk = ROWS * BUCKET_SIZE
    out = pl.pallas_call(
        _body,
        grid=(bh, T_SPLIT),
        in_specs=[
            pl.BlockSpec((1, tchunk, 128), lambda i, j: (i, j, 0)),
            pl.BlockSpec((1, tchunk, 128), lambda i, j: (i, j, 0)),
            pl.BlockSpec((1, 1, DIM, MAX_BUCKETS), lambda i, j: (0, i % HEADS, 0, 0)),
        ],
        out_specs=pl.BlockSpec((1, ROWS, MAX_BUCKETS), lambda i, j: (i, j, 0)),
        out_shape=jax.ShapeDtypeStruct((bh, MAX_BUCKETS, MAX_BUCKETS), jnp.float32),
    )(q, k, linear)
    return out


# speedup vs baseline: 1.6620x; 1.6620x over previous
import jax
import jax.numpy as jnp
from jax.experimental import pallas as pl

HEADS = 32
BUCKET_SIZE = 64
MAX_BUCKETS = 64
DIM = 256
TEMPERATURE = 0.7


BH_BLOCK = 2


def _body(q_ref, k_ref, w_ref, o_ref):
    for b in range(BH_BLOCK):
        qs = jnp.sum(q_ref[b].reshape(MAX_BUCKETS, BUCKET_SIZE, 128), axis=1)
        ks = jnp.sum(k_ref[b].reshape(MAX_BUCKETS, BUCKET_SIZE, 128), axis=1)
        w = w_ref[0, b]
        r = jnp.dot(qs, w[:128, :], preferred_element_type=jnp.float32)
        r = r + jnp.dot(ks, w[128:, :], preferred_element_type=jnp.float32)
        r = jnp.maximum(r, 0.0)

        m = jnp.max(r, axis=-1, keepdims=True)
        iota = jax.lax.broadcasted_iota(jnp.int32, r.shape, 1)
        idx = jnp.min(jnp.where(r == m, iota, MAX_BUCKETS), axis=-1, keepdims=True)
        denom = jnp.sum(jnp.exp((r - m) / TEMPERATURE), axis=-1, keepdims=True)
        val = 1.0 / denom
        o_ref[b] = jnp.where(iota == idx, val, 0.0)


def kernel(q, k, linear, topk):
    bh = q.shape[0]
    out = pl.pallas_call(
        _body,
        grid=(bh // BH_BLOCK,),
        in_specs=[
            pl.BlockSpec((BH_BLOCK, 4096, 128), lambda i: (i, 0, 0)),
            pl.BlockSpec((BH_BLOCK, 4096, 128), lambda i: (i, 0, 0)),
            pl.BlockSpec((1, BH_BLOCK, DIM, MAX_BUCKETS),
                         lambda i: (0, i % (HEADS // BH_BLOCK), 0, 0)),
        ],
        out_specs=pl.BlockSpec((BH_BLOCK, MAX_BUCKETS, MAX_BUCKETS), lambda i: (i, 0, 0)),
        out_shape=jax.ShapeDtypeStruct((bh, MAX_BUCKETS, MAX_BUCKETS), jnp.float32),
    )(q, k, linear)
    return out


# speedup vs baseline: 1.6768x; 1.0089x over previous
import jax
import jax.numpy as jnp
from jax.experimental import pallas as pl

HEADS = 32
BUCKET_SIZE = 64
MAX_BUCKETS = 64
DIM = 256
TEMPERATURE = 0.7


BH_BLOCK = 4


def _body(q_ref, k_ref, w_ref, o_ref):
    for b in range(BH_BLOCK):
        qs = jnp.sum(q_ref[b].reshape(MAX_BUCKETS, BUCKET_SIZE, 128), axis=1)
        ks = jnp.sum(k_ref[b].reshape(MAX_BUCKETS, BUCKET_SIZE, 128), axis=1)
        w = w_ref[0, b]
        r = jnp.dot(qs, w[:128, :], preferred_element_type=jnp.float32)
        r = r + jnp.dot(ks, w[128:, :], preferred_element_type=jnp.float32)
        r = jnp.maximum(r, 0.0)

        m = jnp.max(r, axis=-1, keepdims=True)
        iota = jax.lax.broadcasted_iota(jnp.int32, r.shape, 1)
        idx = jnp.min(jnp.where(r == m, iota, MAX_BUCKETS), axis=-1, keepdims=True)
        denom = jnp.sum(jnp.exp((r - m) / TEMPERATURE), axis=-1, keepdims=True)
        val = 1.0 / denom
        o_ref[b] = jnp.where(iota == idx, val, 0.0)


def kernel(q, k, linear, topk):
    bh = q.shape[0]
    out = pl.pallas_call(
        _body,
        grid=(bh // BH_BLOCK,),
        in_specs=[
            pl.BlockSpec((BH_BLOCK, 4096, 128), lambda i: (i, 0, 0)),
            pl.BlockSpec((BH_BLOCK, 4096, 128), lambda i: (i, 0, 0)),
            pl.BlockSpec((1, BH_BLOCK, DIM, MAX_BUCKETS),
                         lambda i: (0, i % (HEADS // BH_BLOCK), 0, 0)),
        ],
        out_specs=pl.BlockSpec((BH_BLOCK, MAX_BUCKETS, MAX_BUCKETS), lambda i: (i, 0, 0)),
        out_shape=jax.ShapeDtypeStruct((bh, MAX_BUCKETS, MAX_BUCKETS), jnp.float32),
    )(q, k, linear)
    return out
